# baseline (speedup 1.0000x reference)
import functools

import jax
import jax.numpy as jnp
from jax import lax
from jax.experimental import pallas as pl
from jax.experimental.pallas import tpu as pltpu
from jax.experimental.pallas import tpu_sc as plsc

NC = 2
NS = 16
NW = NC * NS
DEGW = 8
DEG_LAG = 8


def _edge_tiling(e):
    if e % NW == 0:
        ept = e // NW
        for ch in range(40, 7, -8):
            if ept % ch == 0:
                cpt = ept // ch
                for nbuf in (10, 5, 4, 6, 3, 2, 1):
                    if cpt % nbuf == 0:
                        return ch, cpt, nbuf, 0
    ch = 128
    cpt = -(-e // (NW * ch))
    if cpt % 4:
        cpt += 4 - cpt % 4
    return ch, cpt, 4, NW * cpt * ch - e


def _sc_mesh():
    return plsc.VectorSubcoreMesh(core_axis_name="c", subcore_axis_name="s")


def _make_deg_kernel(n, n_acc, ch, cpt, r):
    @functools.partial(
        pl.kernel,
        out_type=jax.ShapeDtypeStruct((n // r, NC, r, DEGW), jnp.float32),
        mesh=_sc_mesh(),
        compiler_params=pltpu.CompilerParams(use_tc_tiling_on_sc=False),
        scratch_types=[
            pltpu.VMEM_SHARED((n_acc, DEGW), jnp.float32),
            pltpu.VMEM((cpt, ch), jnp.int32),
            pltpu.VMEM((ch, DEGW), jnp.float32),
            pltpu.SemaphoreType.DMA,
        ],
    )
    def deg_kernel(dst_hbm, zeros_hbm, ones_hbm, out_hbm, acc, dbuf, ones_v,
                   sem):
        c = lax.axis_index("c")
        s = lax.axis_index("s")
        w = c * NS + s
        rpt = n // NS
        pltpu.sync_copy(zeros_hbm.at[pl.ds(s * rpt, rpt)],
                        acc.at[pl.ds(s * rpt, rpt)])
        pltpu.sync_copy(ones_hbm, ones_v)
        pltpu.sync_copy(dst_hbm.at[w], dbuf)
        plsc.subcore_barrier()

        def issue(j, carry):
            pltpu.async_copy(ones_v, acc.at[dbuf.at[j]], sem, add=True)

            @pl.when(j >= DEG_LAG)
            def _():
                pltpu.make_async_copy(ones_v, acc.at[dbuf.at[0]], sem).wait()

            return carry

        lax.fori_loop(0, cpt, issue, 0)

        def drain(j, carry):
            pltpu.make_async_copy(ones_v, acc.at[dbuf.at[0]], sem).wait()
            return carry

        lax.fori_loop(0, DEG_LAG, drain, 0)
        plsc.subcore_barrier()

        @pl.when(s < n // r)
        def _():
            pltpu.sync_copy(acc.at[pl.ds(s * r, r)], out_hbm.at[s, c])

    return deg_kernel


def _make_scatter_kernel(n, n_acc, d, ch, cpt, nbuf, r):
    @functools.partial(
        pl.kernel,
        out_type=jax.ShapeDtypeStruct((n // r, NC, r, d), jnp.float32),
        mesh=_sc_mesh(),
        compiler_params=pltpu.CompilerParams(use_tc_tiling_on_sc=False),
        scratch_types=(
            [
                pltpu.VMEM_SHARED((n_acc, d), jnp.float32),
                pltpu.VMEM_SHARED((n, d), jnp.float32),
                pltpu.VMEM((cpt, ch), jnp.int32),
                pltpu.VMEM((cpt, ch), jnp.int32),
            ]
            + [pltpu.VMEM((ch, d), jnp.float32)] * nbuf
            + [pltpu.SemaphoreType.DMA] * (2 * nbuf)
        ),
    )
    def scatter_kernel(g_hbm, src_hbm, dst_hbm, out_hbm,
                       acc, gtab, sbuf, dbuf, *bufs_and_sems):
        rows = bufs_and_sems[:nbuf]
        gsem = bufs_and_sems[nbuf:2 * nbuf]
        ssem = bufs_and_sems[2 * nbuf:]
        c = lax.axis_index("c")
        s = lax.axis_index("s")
        w = c * NS + s
        rpt = n // NS
        sl = pl.ds(s * rpt, rpt)
        pltpu.sync_copy(g_hbm.at[sl], gtab.at[sl])
        pltpu.sync_copy(g_hbm.at[sl], acc.at[sl])
        pltpu.sync_copy(src_hbm.at[w], sbuf)
        pltpu.sync_copy(dst_hbm.at[w], dbuf)
        plsc.subcore_barrier()

        def gstart(j, b):
            pltpu.async_copy(gtab.at[sbuf.at[j]], rows[b], gsem[b])

        def gwait(b):
            pltpu.make_async_copy(gtab.at[sbuf.at[0]], rows[b],
                                  gsem[b]).wait()

        def sstart(j, b):
            pltpu.async_copy(rows[b], acc.at[dbuf.at[j]], ssem[b], add=True)

        def swait(b):
            pltpu.make_async_copy(rows[b], acc.at[dbuf.at[0]],
                                  ssem[b]).wait()

        for b in range(nbuf):
            gstart(b, b)

        def body(jb, carry):
            j0 = jb * nbuf
            for b in range(nbuf):
                gwait(b)
                sstart(j0 + b, b)
            for b in range(nbuf):
                swait(b)
                nxt = j0 + nbuf + b

                @pl.when(nxt < cpt)
                def _():
                    gstart(nxt, b)

            return carry

        lax.fori_loop(0, cpt // nbuf, body, 0)
        plsc.subcore_barrier()

        @pl.when(s < n // r)
        def _():
            pltpu.sync_copy(acc.at[pl.ds(s * r, r)], out_hbm.at[s, c])

    return scatter_kernel


def _tc_mm_body(x_ref, w1_ref, u_ref):
    u_ref[...] = jnp.dot(x_ref[...], w1_ref[...],
                         preferred_element_type=jnp.float32)


def _tc1_body(u_ref, degp_ref, g_ref, dinv_ref):
    p = degp_ref[...]
    deg = p[0, 0, :, 0:1] + p[0, 1, :, 0:1] + 1.0
    dinv = lax.rsqrt(deg)
    g_ref[...] = u_ref[...] * dinv
    dinv_ref[...] = dinv


def _tc_mid_body(sp_ref, g_ref, dinv_ref, b_ref, w_ref, out_ref):
    p = sp_ref[...]
    comb = p[0, 0] + p[0, 1] - g_ref[...]
    h = jnp.maximum(comb * dinv_ref[...] + b_ref[...], 0.0)
    out_ref[...] = jnp.dot(h, w_ref[...],
                           preferred_element_type=jnp.float32) * dinv_ref[...]


def _tc_out_body(sp_ref, g_ref, dinv_ref, b_ref, wfc_ref, bfc_ref, out_ref):
    p = sp_ref[...]
    comb = p[0, 0] + p[0, 1] - g_ref[...]
    h = jnp.maximum(comb * dinv_ref[...] + b_ref[...], 0.0)
    z = jnp.dot(h, wfc_ref[...], preferred_element_type=jnp.float32)
    out_ref[...] = jax.nn.sigmoid(z + bfc_ref[...])


def _stacked_spec(r, d):
    return pl.BlockSpec((1, NC, r, d), lambda i: (i, 0, 0, 0))


def kernel(x, edge_index, W1, b1, W2, b2, Wfc, bfc):
    n, d_in = x.shape
    e = edge_index.shape[1]
    d_hid = W1.shape[1]
    d_out = W2.shape[1]

    ch, cpt, nbuf, pad = _edge_tiling(e)
    n_acc = n + (NS if pad else 0)

    src = edge_index[0].astype(jnp.int32)
    dst = edge_index[1].astype(jnp.int32)
    if pad:
        src = jnp.concatenate([src, jnp.zeros((pad,), jnp.int32)])
        dst = jnp.concatenate([dst, jnp.full((pad,), n, jnp.int32)])
    src_t = src.reshape(NW, cpt, ch)
    dst_t = dst.reshape(NW, cpt, ch)

    zeros_h = jnp.zeros((n, DEGW), jnp.float32)
    ones_h = jnp.ones((ch, DEGW), jnp.float32)

    r = 5000
    grid = (n // r,)
    u1 = pl.pallas_call(
        _tc_mm_body,
        grid=grid,
        in_specs=[
            pl.BlockSpec((r, d_in), lambda i: (i, 0)),
            pl.BlockSpec((d_in, d_hid), lambda i: (0, 0)),
        ],
        out_specs=pl.BlockSpec((r, d_hid), lambda i: (i, 0)),
        out_shape=jax.ShapeDtypeStruct((n, d_hid), jnp.float32),
    )(x, W1)

    degp = _make_deg_kernel(n, n_acc, ch, cpt, r)(dst_t, zeros_h, ones_h)

    g1, dinv = pl.pallas_call(
        _tc1_body,
        grid=grid,
        in_specs=[
            pl.BlockSpec((r, d_hid), lambda i: (i, 0)),
            _stacked_spec(r, DEGW),
        ],
        out_specs=[
            pl.BlockSpec((r, d_hid), lambda i: (i, 0)),
            pl.BlockSpec((r, 1), lambda i: (i, 0)),
        ],
        out_shape=[
            jax.ShapeDtypeStruct((n, d_hid), jnp.float32),
            jax.ShapeDtypeStruct((n, 1), jnp.float32),
        ],
    )(u1, degp)

    s1 = _make_scatter_kernel(n, n_acc, d_hid, ch, cpt, nbuf, r)(
        g1, src_t, dst_t)

    g2 = pl.pallas_call(
        _tc_mid_body,
        grid=grid,
        in_specs=[
            _stacked_spec(r, d_hid),
            pl.BlockSpec((r, d_hid), lambda i: (i, 0)),
            pl.BlockSpec((r, 1), lambda i: (i, 0)),
            pl.BlockSpec((1, d_hid), lambda i: (0, 0)),
            pl.BlockSpec((d_hid, d_out), lambda i: (0, 0)),
        ],
        out_specs=pl.BlockSpec((r, d_out), lambda i: (i, 0)),
        out_shape=jax.ShapeDtypeStruct((n, d_out), jnp.float32),
    )(s1, g1, dinv, b1.reshape(1, d_hid), W2)

    s2 = _make_scatter_kernel(n, n_acc, d_out, ch, cpt, nbuf, r)(
        g2, src_t, dst_t)

    out = pl.pallas_call(
        _tc_out_body,
        grid=grid,
        in_specs=[
            _stacked_spec(r, d_out),
            pl.BlockSpec((r, d_out), lambda i: (i, 0)),
            pl.BlockSpec((r, 1), lambda i: (i, 0)),
            pl.BlockSpec((1, d_out), lambda i: (0, 0)),
            pl.BlockSpec((d_out, 1), lambda i: (0, 0)),
            pl.BlockSpec((1, 1), lambda i: (0, 0)),
        ],
        out_specs=pl.BlockSpec((r, 1), lambda i: (i, 0)),
        out_shape=jax.ShapeDtypeStruct((n, 1), jnp.float32),
    )(s2, g2, dinv, b2.reshape(1, d_out), Wfc, bfc.reshape(1, 1))

    return out


# speedup vs baseline: 1.0059x; 1.0059x over previous
import functools

import jax
import jax.numpy as jnp
from jax import lax
from jax.experimental import pallas as pl
from jax.experimental.pallas import tpu as pltpu
from jax.experimental.pallas import tpu_sc as plsc

NC = 2
NS = 16
NW = NC * NS
DEGW = 8
DEG_LAG = 16


def _edge_tiling(e):
    if e % NW == 0:
        ept = e // NW
        for ch in range(40, 7, -8):
            if ept % ch == 0:
                cpt = ept // ch
                for nbuf in (5, 4, 6, 3, 2, 1):
                    if cpt % nbuf == 0:
                        return ch, cpt, nbuf, 0
    ch = 128
    cpt = -(-e // (NW * ch))
    if cpt % 4:
        cpt += 4 - cpt % 4
    return ch, cpt, 4, NW * cpt * ch - e


def _sc_mesh():
    return plsc.VectorSubcoreMesh(core_axis_name="c", subcore_axis_name="s")


def _make_deg_kernel(n, n_acc, ch, cpt, r):
    @functools.partial(
        pl.kernel,
        out_type=jax.ShapeDtypeStruct((n // r, NC, r, DEGW), jnp.float32),
        mesh=_sc_mesh(),
        compiler_params=pltpu.CompilerParams(use_tc_tiling_on_sc=False),
        scratch_types=[
            pltpu.VMEM_SHARED((n_acc, DEGW), jnp.float32),
            pltpu.VMEM((cpt, ch), jnp.int32),
            pltpu.VMEM((ch, DEGW), jnp.float32),
            pltpu.SemaphoreType.DMA,
        ],
    )
    def deg_kernel(dst_hbm, zeros_hbm, ones_hbm, out_hbm, acc, dbuf, ones_v,
                   sem):
        c = lax.axis_index("c")
        s = lax.axis_index("s")
        w = c * NS + s
        rpt = n // NS
        pltpu.sync_copy(zeros_hbm.at[pl.ds(s * rpt, rpt)],
                        acc.at[pl.ds(s * rpt, rpt)])
        pltpu.sync_copy(ones_hbm, ones_v)
        pltpu.sync_copy(dst_hbm.at[w], dbuf)
        plsc.subcore_barrier()

        def issue(j, carry):
            pltpu.async_copy(ones_v, acc.at[dbuf.at[j]], sem, add=True)

            @pl.when(j >= DEG_LAG)
            def _():
                pltpu.make_async_copy(ones_v, acc.at[dbuf.at[0]], sem).wait()

            return carry

        lax.fori_loop(0, cpt, issue, 0)

        def drain(j, carry):
            pltpu.make_async_copy(ones_v, acc.at[dbuf.at[0]], sem).wait()
            return carry

        lax.fori_loop(0, DEG_LAG, drain, 0)
        plsc.subcore_barrier()

        @pl.when(s < n // r)
        def _():
            pltpu.sync_copy(acc.at[pl.ds(s * r, r)], out_hbm.at[s, c])

    return deg_kernel


def _make_scatter_kernel(n, n_acc, d, ch, cpt, nbuf, r):
    @functools.partial(
        pl.kernel,
        out_type=jax.ShapeDtypeStruct((n // r, NC, r, d), jnp.float32),
        mesh=_sc_mesh(),
        compiler_params=pltpu.CompilerParams(use_tc_tiling_on_sc=False),
        scratch_types=(
            [
                pltpu.VMEM_SHARED((n_acc, d), jnp.float32),
                pltpu.VMEM_SHARED((n, d), jnp.float32),
                pltpu.VMEM((cpt, ch), jnp.int32),
                pltpu.VMEM((cpt, ch), jnp.int32),
            ]
            + [pltpu.VMEM((ch, d), jnp.float32)] * nbuf
            + [pltpu.SemaphoreType.DMA] * (2 * nbuf)
        ),
    )
    def scatter_kernel(g_hbm, src_hbm, dst_hbm, out_hbm,
                       acc, gtab, sbuf, dbuf, *bufs_and_sems):
        rows = bufs_and_sems[:nbuf]
        gsem = bufs_and_sems[nbuf:2 * nbuf]
        ssem = bufs_and_sems[2 * nbuf:]
        c = lax.axis_index("c")
        s = lax.axis_index("s")
        w = c * NS + s
        rpt = n // NS
        sl = pl.ds(s * rpt, rpt)
        pltpu.sync_copy(g_hbm.at[sl], gtab.at[sl])
        pltpu.sync_copy(g_hbm.at[sl], acc.at[sl])
        pltpu.sync_copy(src_hbm.at[w], sbuf)
        pltpu.sync_copy(dst_hbm.at[w], dbuf)
        plsc.subcore_barrier()

        def gstart(j, b):
            pltpu.async_copy(gtab.at[sbuf.at[j]], rows[b], gsem[b])

        def gwait(b):
            pltpu.make_async_copy(gtab.at[sbuf.at[0]], rows[b],
                                  gsem[b]).wait()

        def sstart(j, b):
            pltpu.async_copy(rows[b], acc.at[dbuf.at[j]], ssem[b], add=True)

        def swait(b):
            pltpu.make_async_copy(rows[b], acc.at[dbuf.at[0]],
                                  ssem[b]).wait()

        for b in range(nbuf):
            gstart(b, b)

        def body(jb, carry):
            j0 = jb * nbuf
            for b in range(nbuf):
                gwait(b)
                sstart(j0 + b, b)
            for b in range(nbuf):
                swait(b)
                nxt = j0 + nbuf + b

                @pl.when(nxt < cpt)
                def _():
                    gstart(nxt, b)

            return carry

        lax.fori_loop(0, cpt // nbuf, body, 0)
        plsc.subcore_barrier()

        @pl.when(s < n // r)
        def _():
            pltpu.sync_copy(acc.at[pl.ds(s * r, r)], out_hbm.at[s, c])

    return scatter_kernel


def _tc_mm_body(x_ref, w1_ref, u_ref):
    u_ref[...] = jnp.dot(x_ref[...], w1_ref[...],
                         preferred_element_type=jnp.float32)


def _tc1_body(u_ref, degp_ref, g_ref, dinv_ref):
    p = degp_ref[...]
    deg = p[0, 0, :, 0:1] + p[0, 1, :, 0:1] + 1.0
    dinv = lax.rsqrt(deg)
    g_ref[...] = u_ref[...] * dinv
    dinv_ref[...] = dinv


def _tc_mid_body(sp_ref, g_ref, dinv_ref, b_ref, w_ref, out_ref):
    p = sp_ref[...]
    comb = p[0, 0] + p[0, 1] - g_ref[...]
    h = jnp.maximum(comb * dinv_ref[...] + b_ref[...], 0.0)
    out_ref[...] = jnp.dot(h, w_ref[...],
                           preferred_element_type=jnp.float32) * dinv_ref[...]


def _tc_out_body(sp_ref, g_ref, dinv_ref, b_ref, wfc_ref, bfc_ref, out_ref):
    p = sp_ref[...]
    comb = p[0, 0] + p[0, 1] - g_ref[...]
    h = jnp.maximum(comb * dinv_ref[...] + b_ref[...], 0.0)
    z = jnp.dot(h, wfc_ref[...], preferred_element_type=jnp.float32)
    out_ref[...] = jax.nn.sigmoid(z + bfc_ref[...])


def _stacked_spec(r, d):
    return pl.BlockSpec((1, NC, r, d), lambda i: (i, 0, 0, 0))


def kernel(x, edge_index, W1, b1, W2, b2, Wfc, bfc):
    n, d_in = x.shape
    e = edge_index.shape[1]
    d_hid = W1.shape[1]
    d_out = W2.shape[1]

    ch, cpt, nbuf, pad = _edge_tiling(e)
    n_acc = n + (NS if pad else 0)

    src = edge_index[0].astype(jnp.int32)
    dst = edge_index[1].astype(jnp.int32)
    if pad:
        src = jnp.concatenate([src, jnp.zeros((pad,), jnp.int32)])
        dst = jnp.concatenate([dst, jnp.full((pad,), n, jnp.int32)])
    src_t = src.reshape(NW, cpt, ch)
    dst_t = dst.reshape(NW, cpt, ch)

    zeros_h = jnp.zeros((n, DEGW), jnp.float32)
    ones_h = jnp.ones((ch, DEGW), jnp.float32)

    r = 5000
    grid = (n // r,)
    u1 = pl.pallas_call(
        _tc_mm_body,
        grid=grid,
        in_specs=[
            pl.BlockSpec((r, d_in), lambda i: (i, 0)),
            pl.BlockSpec((d_in, d_hid), lambda i: (0, 0)),
        ],
        out_specs=pl.BlockSpec((r, d_hid), lambda i: (i, 0)),
        out_shape=jax.ShapeDtypeStruct((n, d_hid), jnp.float32),
    )(x, W1)

    degp = _make_deg_kernel(n, n_acc, ch, cpt, r)(dst_t, zeros_h, ones_h)

    g1, dinv = pl.pallas_call(
        _tc1_body,
        grid=grid,
        in_specs=[
            pl.BlockSpec((r, d_hid), lambda i: (i, 0)),
            _stacked_spec(r, DEGW),
        ],
        out_specs=[
            pl.BlockSpec((r, d_hid), lambda i: (i, 0)),
            pl.BlockSpec((r, 1), lambda i: (i, 0)),
        ],
        out_shape=[
            jax.ShapeDtypeStruct((n, d_hid), jnp.float32),
            jax.ShapeDtypeStruct((n, 1), jnp.float32),
        ],
    )(u1, degp)

    s1 = _make_scatter_kernel(n, n_acc, d_hid, ch, cpt, nbuf, r)(
        g1, src_t, dst_t)

    g2 = pl.pallas_call(
        _tc_mid_body,
        grid=grid,
        in_specs=[
            _stacked_spec(r, d_hid),
            pl.BlockSpec((r, d_hid), lambda i: (i, 0)),
            pl.BlockSpec((r, 1), lambda i: (i, 0)),
            pl.BlockSpec((1, d_hid), lambda i: (0, 0)),
            pl.BlockSpec((d_hid, d_out), lambda i: (0, 0)),
        ],
        out_specs=pl.BlockSpec((r, d_out), lambda i: (i, 0)),
        out_shape=jax.ShapeDtypeStruct((n, d_out), jnp.float32),
    )(s1, g1, dinv, b1.reshape(1, d_hid), W2)

    s2 = _make_scatter_kernel(n, n_acc, d_out, ch, cpt, nbuf, r)(
        g2, src_t, dst_t)

    out = pl.pallas_call(
        _tc_out_body,
        grid=grid,
        in_specs=[
            _stacked_spec(r, d_out),
            pl.BlockSpec((r, d_out), lambda i: (i, 0)),
            pl.BlockSpec((r, 1), lambda i: (i, 0)),
            pl.BlockSpec((1, d_out), lambda i: (0, 0)),
            pl.BlockSpec((d_out, 1), lambda i: (0, 0)),
            pl.BlockSpec((1, 1), lambda i: (0, 0)),
        ],
        out_specs=pl.BlockSpec((r, 1), lambda i: (i, 0)),
        out_shape=jax.ShapeDtypeStruct((n, 1), jnp.float32),
    )(s2, g2, dinv, b2.reshape(1, d_out), Wfc, bfc.reshape(1, 1))

    return out
